# baseline (speedup 1.0000x reference)
import functools

import jax
import jax.numpy as jnp
from jax import lax
from jax.experimental import pallas as pl
from jax.experimental.pallas import tpu as pltpu
from jax.experimental.pallas import tpu_sc as plsc

NC = 2
NS = 16
NW = NC * NS

BN = 4000


def _leaky(z):
    return jnp.where(z > 0, z, 0.01 * z)


K = 12


def _make_segsum(ng, n, rows_pad):
    split_groups = ng > 1
    rpt = rows_pad // (NS if split_groups else NW)
    nblk = rpt // K
    sp_z = n // NS
    zch = next(k for k in range(8, 512) if sp_z % k == 0 and sp_z // k <= 128)
    zrows = sp_z // zch
    out_shape = ((ng, n, 16) if split_groups else (NC, ng, n, 16))

    mesh = plsc.VectorSubcoreMesh(core_axis_name="c", subcore_axis_name="s")

    @functools.partial(
        pl.kernel,
        out_type=jax.ShapeDtypeStruct(out_shape, jnp.float32),
        mesh=mesh,
        compiler_params=pltpu.CompilerParams(use_tc_tiling_on_sc=False),
        scratch_types=[
            pltpu.VMEM((K, 128), jnp.int32),
            pltpu.VMEM((K, 128), jnp.int32),
            pltpu.VMEM((K, 128, 16), jnp.float32),
            pltpu.VMEM((zrows, 16), jnp.float32),
            pltpu.VMEM_SHARED((n, 16), jnp.float32),
            pltpu.SemaphoreType.DMA,
            pltpu.SemaphoreType.DMA,
        ],
    )
    def segsum(*args):
        src_hbm, dst_hbm = args[0], args[1]
        tabs = args[2:2 + ng]
        zeros_hbm = args[2 + ng]
        out_hbm = args[3 + ng]
        srcv0, dstv0, rows0, zbuf, aggr, semg0, sems = args[4 + ng:]

        cid = lax.axis_index("c")
        sid = lax.axis_index("s")
        if split_groups:
            base = sid * rpt
        else:
            base = (sid * NC + cid) * rpt

        pltpu.sync_copy(zeros_hbm, zbuf)

        def load_idx(rb, sv, dv):
            pltpu.sync_copy(src_hbm.at[pl.ds(rb, K)], sv)
            pltpu.sync_copy(dst_hbm.at[pl.ds(rb, K)], dv)

        def run_group(g):
            tab = tabs[g]

            zb = sid * sp_z
            for k in range(zch):
                pltpu.sync_copy(zbuf, aggr.at[pl.ds(zb + k * zrows, zrows)])
            plsc.subcore_barrier()

            def body(j, carry):
                rb = base + j * K
                load_idx(rb, srcv0, dstv0)
                cps = [
                    pltpu.async_copy(tab.at[srcv0.at[jj]], rows0.at[jj], semg0)
                    for jj in range(K)
                ]
                scs = []
                for jj in range(K):
                    cps[jj].wait()
                    scs.append(
                        pltpu.async_copy(rows0.at[jj], aggr.at[dstv0.at[jj]],
                                         sems, add=True))
                for sc in scs:
                    sc.wait()
                return carry

            lax.fori_loop(0, nblk, body, 0)
            plsc.subcore_barrier()

            if split_groups:
                dst = out_hbm.at[g, pl.ds(sid * sp_z, sp_z)]
            else:
                dst = out_hbm.at[cid, g, pl.ds(sid * sp_z, sp_z)]
            pltpu.sync_copy(aggr.at[pl.ds(sid * sp_z, sp_z)], dst)
            plsc.subcore_barrier()

        if split_groups:
            gpc = ng // NC
            for g in range(ng):
                pl.when(cid == g // gpc)(functools.partial(run_group, g))
        else:
            for g in range(ng):
                run_group(g)

    return segsum


def _table_specs(n):
    nb = n // BN
    outs = [pl.BlockSpec((BN, 16), lambda i: (i, 0)) for _ in range(4)]
    shapes = [jax.ShapeDtypeStruct((n + BN, 16), jnp.float32)
              for _ in range(4)]
    return nb, outs, shapes


def _write_tables(i, nb, h, orefs):
    @pl.when(i < nb)
    def _():
        for g in range(4):
            orefs[g][...] = h[:, 16 * g:16 * (g + 1)]

    @pl.when(i == nb)
    def _():
        for g in range(4):
            orefs[g][...] = jnp.zeros((BN, 16), jnp.float32)


def _k1_body(p_ref, x_ref, a_ref, w_ref, b_ref, *orefs):
    i = pl.program_id(0)
    nb = pl.num_programs(0) - 1
    deg = p_ref[0, 0, :, 0] + p_ref[1, 0, :, 0]
    z = (deg[:, None] * a_ref[0][None, :]
         + x_ref[:, 0][:, None] * w_ref[0][None, :]
         + b_ref[0][None, :])
    _write_tables(i, nb, _leaky(z), orefs)


def _layer0(p0, xr, a, w, b, n):
    nb, outs, shapes = _table_specs(n)
    cl = lambda i: jnp.minimum(i, nb - 1)
    return pl.pallas_call(
        _k1_body,
        grid=(nb + 1,),
        in_specs=[
            pl.BlockSpec((NC, 1, BN, 16), lambda i: (0, 0, cl(i), 0)),
            pl.BlockSpec((BN, 1), lambda i: (cl(i), 0)),
            pl.BlockSpec((1, 64), lambda i: (0, 0)),
            pl.BlockSpec((1, 64), lambda i: (0, 0)),
            pl.BlockSpec((1, 64), lambda i: (0, 0)),
        ],
        out_specs=outs,
        out_shape=shapes,
    )(p0, xr, a, w, b)


def _k2_body(p_ref, h0, h1, h2, h3, wr_ref, wt_ref, b_ref, *orefs):
    i = pl.program_id(0)
    nb = pl.num_programs(0) - 1
    aggr = jnp.concatenate([p_ref[g] for g in range(4)], axis=1)
    h = jnp.concatenate([h0[...], h1[...], h2[...], h3[...]], axis=1)
    z = (jnp.dot(aggr, wr_ref[...].T, preferred_element_type=jnp.float32)
         + jnp.dot(h, wt_ref[...].T, preferred_element_type=jnp.float32)
         + b_ref[...])
    _write_tables(i, nb, _leaky(z), orefs)


def _conv(p, htabs, wr, wt, b, n):
    nb, outs, shapes = _table_specs(n)
    cl = lambda i: jnp.minimum(i, nb - 1)
    return pl.pallas_call(
        _k2_body,
        grid=(nb + 1,),
        in_specs=[
            pl.BlockSpec((4, BN, 16), lambda i: (0, cl(i), 0)),
            pl.BlockSpec((BN, 16), lambda i: (cl(i), 0)),
            pl.BlockSpec((BN, 16), lambda i: (cl(i), 0)),
            pl.BlockSpec((BN, 16), lambda i: (cl(i), 0)),
            pl.BlockSpec((BN, 16), lambda i: (cl(i), 0)),
            pl.BlockSpec(wr.shape, lambda i: (0, 0)),
            pl.BlockSpec(wt.shape, lambda i: (0, 0)),
            pl.BlockSpec(b.shape, lambda i: (0, 0)),
        ],
        out_specs=outs,
        out_shape=shapes,
    )(p, *htabs, wr, wt, b)


def _make_k4_body(n):
    def _k4_body(p_ref, h0, h1, h2, h3ref_, wr_ref, wt_ref, b_ref, fw1_ref,
                 fb1_ref, fw2_ref, fb2_ref, h3_ref, maxv, maxi, sumv, colsum):
        i = pl.program_id(0)
        aggr = jnp.concatenate([p_ref[g] for g in range(4)], axis=1)
        h = jnp.concatenate([h0[...], h1[...], h2[...], h3ref_[...]], axis=1)
        h3 = _leaky(
            jnp.dot(aggr, wr_ref[...].T, preferred_element_type=jnp.float32)
            + jnp.dot(h, wt_ref[...].T, preferred_element_type=jnp.float32)
            + b_ref[...])
        h3_ref[...] = h3
        t = _leaky(jnp.dot(h3, fw1_ref[...].T, preferred_element_type=jnp.float32)
                   + fb1_ref[...])
        q = jax.nn.sigmoid(
            jnp.dot(t, fw2_ref[...].T, preferred_element_type=jnp.float32)[:, 0]
            + fb2_ref[0, 0])
        gidx = i * BN + lax.broadcasted_iota(jnp.int32, (BN, 1), 0)[:, 0]
        valid = gidx < (n - 1)
        qm = jnp.where(valid, q, -jnp.inf)
        bmax = jnp.max(qm)
        barg = jnp.min(jnp.where(qm == bmax, gidx, jnp.int32(2 ** 30)))
        bsum = jnp.sum(jnp.where(valid, q, 0.0))
        csum = jnp.sum(h3, axis=0)

        @pl.when(i == 0)
        def _():
            maxv[0, 0] = bmax
            maxi[0, 0] = barg
            sumv[0, 0] = bsum
            colsum[...] = csum.reshape(1, 32)

        @pl.when(i > 0)
        def _():
            old = maxv[0, 0]
            take = bmax > old
            maxv[0, 0] = jnp.where(take, bmax, old)
            maxi[0, 0] = jnp.where(take, barg, maxi[0, 0])
            sumv[0, 0] = sumv[0, 0] + bsum
            colsum[...] = colsum[...] + csum.reshape(1, 32)

    return _k4_body


def _conv_last(p, htabs, wr, wt, b, fw1, fb1, fw2, fb2, n):
    return pl.pallas_call(
        _make_k4_body(n),
        grid=(n // BN,),
        in_specs=[
            pl.BlockSpec((4, BN, 16), lambda i: (0, i, 0)),
            pl.BlockSpec((BN, 16), lambda i: (i, 0)),
            pl.BlockSpec((BN, 16), lambda i: (i, 0)),
            pl.BlockSpec((BN, 16), lambda i: (i, 0)),
            pl.BlockSpec((BN, 16), lambda i: (i, 0)),
            pl.BlockSpec((32, 64), lambda i: (0, 0)),
            pl.BlockSpec((32, 64), lambda i: (0, 0)),
            pl.BlockSpec((1, 32), lambda i: (0, 0)),
            pl.BlockSpec((64, 32), lambda i: (0, 0)),
            pl.BlockSpec((1, 64), lambda i: (0, 0)),
            pl.BlockSpec((1, 64), lambda i: (0, 0)),
            pl.BlockSpec(memory_space=pltpu.SMEM),
        ],
        out_specs=[
            pl.BlockSpec((BN, 32), lambda i: (i, 0)),
            pl.BlockSpec(memory_space=pltpu.SMEM),
            pl.BlockSpec(memory_space=pltpu.SMEM),
            pl.BlockSpec(memory_space=pltpu.SMEM),
            pl.BlockSpec((1, 32), lambda i: (0, 0)),
        ],
        out_shape=[
            jax.ShapeDtypeStruct((n, 32), jnp.float32),
            jax.ShapeDtypeStruct((1, 1), jnp.float32),
            jax.ShapeDtypeStruct((1, 1), jnp.int32),
            jax.ShapeDtypeStruct((1, 1), jnp.float32),
            jax.ShapeDtypeStruct((1, 32), jnp.float32),
        ],
    )(p, *htabs, wr, wt, b, fw1, fb1, fw2, fb2)


def _k5_body(sidx_ref, h3_ref, hf_ref, c_ref, m1_ref, s1_ref, w1_ref, b1_ref,
             w2_ref, b2_ref, tw1_ref, tb1_ref, tw2_ref, tb2_ref,
             out_ref, m2_ref, s2_ref):
    i = pl.program_id(0)
    ng = pl.num_programs(0)
    h3 = h3_ref[...]
    hf = hf_ref[pl.ds(sidx_ref[0] % 8, 1), :]
    w1 = w1_ref[...]
    z = (jnp.dot(h3, w1[:, :32].T, preferred_element_type=jnp.float32)
         + jnp.dot(hf, w1[:, 32:].T, preferred_element_type=jnp.float32)
         + b1_ref[...])
    q = jax.nn.sigmoid(
        jnp.dot(_leaky(z), w2_ref[...].T, preferred_element_type=jnp.float32)[:, 0]
        + b2_ref[0, 0])
    bmax = jnp.max(q)
    bsum = jnp.sum(q)

    @pl.when(i == 0)
    def _():
        m2_ref[0, 0] = bmax
        s2_ref[0, 0] = bsum

    @pl.when(i > 0)
    def _():
        m2_ref[0, 0] = jnp.maximum(m2_ref[0, 0], bmax)
        s2_ref[0, 0] = s2_ref[0, 0] + bsum

    @pl.when(i == ng - 1)
    def _():
        t3 = _leaky(
            jnp.dot(c_ref[...], tw1_ref[...].T, preferred_element_type=jnp.float32)
            + tb1_ref[...])
        q3 = jax.nn.sigmoid(
            jnp.dot(t3, tw2_ref[...].T, preferred_element_type=jnp.float32)
            + tb2_ref[...])
        p3m = jnp.max(q3)
        p3s = jnp.sum(q3)
        out_ref[0, 0] = ((m1_ref[0, 0] / s1_ref[0, 0])
                         * (m2_ref[0, 0] / s2_ref[0, 0])
                         * (p3m / p3s))


def _heads23(sidx, h3, colsum, maxv, sumv, w1, b1, w2, b2, tw1, tb1, tw2, tb2, n):
    grid_spec = pltpu.PrefetchScalarGridSpec(
        num_scalar_prefetch=1,
        grid=(n // BN,),
        in_specs=[
            pl.BlockSpec((BN, 32), lambda i, s: (i, 0)),
            pl.BlockSpec((8, 32), lambda i, s: (s[0] // 8, 0)),
            pl.BlockSpec((1, 32), lambda i, s: (0, 0)),
            pl.BlockSpec(memory_space=pltpu.SMEM),
            pl.BlockSpec(memory_space=pltpu.SMEM),
            pl.BlockSpec((64, 64), lambda i, s: (0, 0)),
            pl.BlockSpec((1, 64), lambda i, s: (0, 0)),
            pl.BlockSpec((1, 64), lambda i, s: (0, 0)),
            pl.BlockSpec(memory_space=pltpu.SMEM),
            pl.BlockSpec((64, 32), lambda i, s: (0, 0)),
            pl.BlockSpec((1, 64), lambda i, s: (0, 0)),
            pl.BlockSpec((2, 64), lambda i, s: (0, 0)),
            pl.BlockSpec((1, 2), lambda i, s: (0, 0)),
        ],
        out_specs=[
            pl.BlockSpec(memory_space=pltpu.SMEM),
            pl.BlockSpec(memory_space=pltpu.SMEM),
            pl.BlockSpec(memory_space=pltpu.SMEM),
        ],
    )
    return pl.pallas_call(
        _k5_body,
        grid_spec=grid_spec,
        out_shape=[
            jax.ShapeDtypeStruct((1, 1), jnp.float32),
            jax.ShapeDtypeStruct((1, 1), jnp.float32),
            jax.ShapeDtypeStruct((1, 1), jnp.float32),
        ],
    )(sidx, h3, h3, colsum, maxv, sumv, w1, b1, w2, b2, tw1, tb1, tw2, tb2)


def kernel(x, edge_index, batch, Wrel0, brel0, Wroot0, Wrel1, brel1, Wroot1,
           Wrel2, brel2, Wroot2, fp_w1, fp_b1, fp_w2, fp_b2, sp_w1, sp_b1,
           sp_w2, sp_b2, tp_w1, tp_b1, tp_w2, tp_b2):
    n = x.shape[0]
    e = edge_index.shape[1]
    assert e % 128 == 0 and n % NS == 0
    rows = e // 128
    rows_pad = NW * K * ((rows + NW * K - 1) // (NW * K))
    pad = rows_pad - rows

    src2d = jnp.concatenate(
        [edge_index[0].reshape(rows, 128),
         jnp.full((pad, 128), n, jnp.int32)], axis=0)
    dst2d = jnp.concatenate(
        [edge_index[1].reshape(rows, 128),
         jnp.zeros((pad, 128), jnp.int32)], axis=0)

    sp_z = n // NS
    zch = next(k for k in range(8, 512) if sp_z % k == 0 and sp_z // k <= 128)
    zeros_in = jnp.zeros((sp_z // zch, 16), jnp.float32)

    seg1 = _make_segsum(1, n, rows_pad)
    seg4 = _make_segsum(4, n, rows_pad)

    x16 = jnp.pad(x, ((0, 1), (0, 15)))
    p0 = seg1(src2d, dst2d, x16, zeros_in)
    h1tabs = _layer0(p0, x, Wrel0.T, Wroot0.T, brel0.reshape(1, 64), n)

    p1 = seg4(src2d, dst2d, *h1tabs, zeros_in)
    h2tabs = _conv(p1, h1tabs, Wrel1, Wroot1, brel1.reshape(1, 64), n)

    p2 = seg4(src2d, dst2d, *h2tabs, zeros_in)
    h3, maxv, maxi, sumv, colsum = _conv_last(
        p2, h2tabs, Wrel2, Wroot2, brel2.reshape(1, 32),
        fp_w1, fp_b1.reshape(1, 64), fp_w2, fp_b2.reshape(1, 1), n)

    out, _, _ = _heads23(
        maxi.reshape(-1), h3, colsum, maxv, sumv,
        sp_w1, sp_b1.reshape(1, 64), sp_w2, sp_b2.reshape(1, 1),
        tp_w1, tp_b1.reshape(1, 64), tp_w2, tp_b2.reshape(1, 2), n)
    return out[0, 0]


# speedup vs baseline: 1.0047x; 1.0047x over previous
import functools

import jax
import jax.numpy as jnp
from jax import lax
from jax.experimental import pallas as pl
from jax.experimental.pallas import tpu as pltpu
from jax.experimental.pallas import tpu_sc as plsc

NC = 2
NS = 16
NW = NC * NS

BN = 4000


def _leaky(z):
    return jnp.where(z > 0, z, 0.01 * z)


K = 8


def _make_segsum(ng, n, rows_pad):
    split_groups = ng > 1
    rpt = rows_pad // (NS if split_groups else NW)
    nblk = rpt // K
    sp_z = n // NS
    zch = next(k for k in range(8, 512) if sp_z % k == 0 and sp_z // k <= 128)
    zrows = sp_z // zch
    out_shape = ((ng, n, 16) if split_groups else (NC, ng, n, 16))

    mesh = plsc.VectorSubcoreMesh(core_axis_name="c", subcore_axis_name="s")

    @functools.partial(
        pl.kernel,
        out_type=jax.ShapeDtypeStruct(out_shape, jnp.float32),
        mesh=mesh,
        compiler_params=pltpu.CompilerParams(use_tc_tiling_on_sc=False),
        scratch_types=[
            pltpu.VMEM((K, 128), jnp.int32),
            pltpu.VMEM((K, 128), jnp.int32),
            pltpu.VMEM((K, 128, 16), jnp.float32),
            pltpu.VMEM((zrows, 16), jnp.float32),
            pltpu.VMEM_SHARED((n, 16), jnp.float32),
            pltpu.SemaphoreType.DMA,
        ],
    )
    def segsum(*args):
        src_hbm, dst_hbm = args[0], args[1]
        tabs = args[2:2 + ng]
        zeros_hbm = args[2 + ng]
        out_hbm = args[3 + ng]
        srcv0, dstv0, rows0, zbuf, aggr, semg0 = args[4 + ng:]

        cid = lax.axis_index("c")
        sid = lax.axis_index("s")
        if split_groups:
            base = sid * rpt
        else:
            base = (sid * NC + cid) * rpt

        pltpu.sync_copy(zeros_hbm, zbuf)

        def load_idx(rb, sv, dv):
            pltpu.sync_copy(src_hbm.at[pl.ds(rb, K)], sv)
            pltpu.sync_copy(dst_hbm.at[pl.ds(rb, K)], dv)

        def run_group(g):
            tab = tabs[g]

            zb = sid * sp_z
            for k in range(zch):
                pltpu.sync_copy(zbuf, aggr.at[pl.ds(zb + k * zrows, zrows)])
            plsc.subcore_barrier()

            def body(j, carry):
                rb = base + j * K
                load_idx(rb, srcv0, dstv0)
                cps = [
                    pltpu.async_copy(tab.at[srcv0.at[jj]], rows0.at[jj], semg0)
                    for jj in range(K)
                ]
                for cp in cps:
                    cp.wait()
                scs = [
                    pltpu.async_copy(rows0.at[jj], aggr.at[dstv0.at[jj]],
                                     semg0, add=True)
                    for jj in range(K)
                ]
                for sc in scs:
                    sc.wait()
                return carry

            lax.fori_loop(0, nblk, body, 0)
            plsc.subcore_barrier()

            if split_groups:
                dst = out_hbm.at[g, pl.ds(sid * sp_z, sp_z)]
            else:
                dst = out_hbm.at[cid, g, pl.ds(sid * sp_z, sp_z)]
            pltpu.sync_copy(aggr.at[pl.ds(sid * sp_z, sp_z)], dst)
            plsc.subcore_barrier()

        if split_groups:
            gpc = ng // NC
            for g in range(ng):
                pl.when(cid == g // gpc)(functools.partial(run_group, g))
        else:
            for g in range(ng):
                run_group(g)

    return segsum


def _table_specs(n):
    nb = n // BN
    outs = [pl.BlockSpec((BN, 16), lambda i: (i, 0)) for _ in range(4)]
    shapes = [jax.ShapeDtypeStruct((n + BN, 16), jnp.float32)
              for _ in range(4)]
    return nb, outs, shapes


def _write_tables(i, nb, h, orefs):
    @pl.when(i < nb)
    def _():
        for g in range(4):
            orefs[g][...] = h[:, 16 * g:16 * (g + 1)]

    @pl.when(i == nb)
    def _():
        for g in range(4):
            orefs[g][...] = jnp.zeros((BN, 16), jnp.float32)


def _k1_body(p_ref, x_ref, a_ref, w_ref, b_ref, *orefs):
    i = pl.program_id(0)
    nb = pl.num_programs(0) - 1
    deg = p_ref[0, 0, :, 0] + p_ref[1, 0, :, 0]
    z = (deg[:, None] * a_ref[0][None, :]
         + x_ref[:, 0][:, None] * w_ref[0][None, :]
         + b_ref[0][None, :])
    _write_tables(i, nb, _leaky(z), orefs)


def _layer0(p0, xr, a, w, b, n):
    nb, outs, shapes = _table_specs(n)
    cl = lambda i: jnp.minimum(i, nb - 1)
    return pl.pallas_call(
        _k1_body,
        grid=(nb + 1,),
        in_specs=[
            pl.BlockSpec((NC, 1, BN, 16), lambda i: (0, 0, cl(i), 0)),
            pl.BlockSpec((BN, 1), lambda i: (cl(i), 0)),
            pl.BlockSpec((1, 64), lambda i: (0, 0)),
            pl.BlockSpec((1, 64), lambda i: (0, 0)),
            pl.BlockSpec((1, 64), lambda i: (0, 0)),
        ],
        out_specs=outs,
        out_shape=shapes,
    )(p0, xr, a, w, b)


def _k2_body(p_ref, h0, h1, h2, h3, wr_ref, wt_ref, b_ref, *orefs):
    i = pl.program_id(0)
    nb = pl.num_programs(0) - 1
    aggr = jnp.concatenate([p_ref[g] for g in range(4)], axis=1)
    h = jnp.concatenate([h0[...], h1[...], h2[...], h3[...]], axis=1)
    z = (jnp.dot(aggr, wr_ref[...].T, preferred_element_type=jnp.float32)
         + jnp.dot(h, wt_ref[...].T, preferred_element_type=jnp.float32)
         + b_ref[...])
    _write_tables(i, nb, _leaky(z), orefs)


def _conv(p, htabs, wr, wt, b, n):
    nb, outs, shapes = _table_specs(n)
    cl = lambda i: jnp.minimum(i, nb - 1)
    return pl.pallas_call(
        _k2_body,
        grid=(nb + 1,),
        in_specs=[
            pl.BlockSpec((4, BN, 16), lambda i: (0, cl(i), 0)),
            pl.BlockSpec((BN, 16), lambda i: (cl(i), 0)),
            pl.BlockSpec((BN, 16), lambda i: (cl(i), 0)),
            pl.BlockSpec((BN, 16), lambda i: (cl(i), 0)),
            pl.BlockSpec((BN, 16), lambda i: (cl(i), 0)),
            pl.BlockSpec(wr.shape, lambda i: (0, 0)),
            pl.BlockSpec(wt.shape, lambda i: (0, 0)),
            pl.BlockSpec(b.shape, lambda i: (0, 0)),
        ],
        out_specs=outs,
        out_shape=shapes,
    )(p, *htabs, wr, wt, b)


def _make_k4_body(n):
    def _k4_body(p_ref, h0, h1, h2, h3ref_, wr_ref, wt_ref, b_ref, fw1_ref,
                 fb1_ref, fw2_ref, fb2_ref, h3_ref, maxv, maxi, sumv, colsum):
        i = pl.program_id(0)
        aggr = jnp.concatenate([p_ref[g] for g in range(4)], axis=1)
        h = jnp.concatenate([h0[...], h1[...], h2[...], h3ref_[...]], axis=1)
        h3 = _leaky(
            jnp.dot(aggr, wr_ref[...].T, preferred_element_type=jnp.float32)
            + jnp.dot(h, wt_ref[...].T, preferred_element_type=jnp.float32)
            + b_ref[...])
        h3_ref[...] = h3
        t = _leaky(jnp.dot(h3, fw1_ref[...].T, preferred_element_type=jnp.float32)
                   + fb1_ref[...])
        q = jax.nn.sigmoid(
            jnp.dot(t, fw2_ref[...].T, preferred_element_type=jnp.float32)[:, 0]
            + fb2_ref[0, 0])
        gidx = i * BN + lax.broadcasted_iota(jnp.int32, (BN, 1), 0)[:, 0]
        valid = gidx < (n - 1)
        qm = jnp.where(valid, q, -jnp.inf)
        bmax = jnp.max(qm)
        barg = jnp.min(jnp.where(qm == bmax, gidx, jnp.int32(2 ** 30)))
        bsum = jnp.sum(jnp.where(valid, q, 0.0))
        csum = jnp.sum(h3, axis=0)

        @pl.when(i == 0)
        def _():
            maxv[0, 0] = bmax
            maxi[0, 0] = barg
            sumv[0, 0] = bsum
            colsum[...] = csum.reshape(1, 32)

        @pl.when(i > 0)
        def _():
            old = maxv[0, 0]
            take = bmax > old
            maxv[0, 0] = jnp.where(take, bmax, old)
            maxi[0, 0] = jnp.where(take, barg, maxi[0, 0])
            sumv[0, 0] = sumv[0, 0] + bsum
            colsum[...] = colsum[...] + csum.reshape(1, 32)

    return _k4_body


def _conv_last(p, htabs, wr, wt, b, fw1, fb1, fw2, fb2, n):
    return pl.pallas_call(
        _make_k4_body(n),
        grid=(n // BN,),
        in_specs=[
            pl.BlockSpec((4, BN, 16), lambda i: (0, i, 0)),
            pl.BlockSpec((BN, 16), lambda i: (i, 0)),
            pl.BlockSpec((BN, 16), lambda i: (i, 0)),
            pl.BlockSpec((BN, 16), lambda i: (i, 0)),
            pl.BlockSpec((BN, 16), lambda i: (i, 0)),
            pl.BlockSpec((32, 64), lambda i: (0, 0)),
            pl.BlockSpec((32, 64), lambda i: (0, 0)),
            pl.BlockSpec((1, 32), lambda i: (0, 0)),
            pl.BlockSpec((64, 32), lambda i: (0, 0)),
            pl.BlockSpec((1, 64), lambda i: (0, 0)),
            pl.BlockSpec((1, 64), lambda i: (0, 0)),
            pl.BlockSpec(memory_space=pltpu.SMEM),
        ],
        out_specs=[
            pl.BlockSpec((BN, 32), lambda i: (i, 0)),
            pl.BlockSpec(memory_space=pltpu.SMEM),
            pl.BlockSpec(memory_space=pltpu.SMEM),
            pl.BlockSpec(memory_space=pltpu.SMEM),
            pl.BlockSpec((1, 32), lambda i: (0, 0)),
        ],
        out_shape=[
            jax.ShapeDtypeStruct((n, 32), jnp.float32),
            jax.ShapeDtypeStruct((1, 1), jnp.float32),
            jax.ShapeDtypeStruct((1, 1), jnp.int32),
            jax.ShapeDtypeStruct((1, 1), jnp.float32),
            jax.ShapeDtypeStruct((1, 32), jnp.float32),
        ],
    )(p, *htabs, wr, wt, b, fw1, fb1, fw2, fb2)


def _k5_body(sidx_ref, h3_ref, hf_ref, c_ref, m1_ref, s1_ref, w1_ref, b1_ref,
             w2_ref, b2_ref, tw1_ref, tb1_ref, tw2_ref, tb2_ref,
             out_ref, m2_ref, s2_ref):
    i = pl.program_id(0)
    ng = pl.num_programs(0)
    h3 = h3_ref[...]
    hf = hf_ref[pl.ds(sidx_ref[0] % 8, 1), :]
    w1 = w1_ref[...]
    z = (jnp.dot(h3, w1[:, :32].T, preferred_element_type=jnp.float32)
         + jnp.dot(hf, w1[:, 32:].T, preferred_element_type=jnp.float32)
         + b1_ref[...])
    q = jax.nn.sigmoid(
        jnp.dot(_leaky(z), w2_ref[...].T, preferred_element_type=jnp.float32)[:, 0]
        + b2_ref[0, 0])
    bmax = jnp.max(q)
    bsum = jnp.sum(q)

    @pl.when(i == 0)
    def _():
        m2_ref[0, 0] = bmax
        s2_ref[0, 0] = bsum

    @pl.when(i > 0)
    def _():
        m2_ref[0, 0] = jnp.maximum(m2_ref[0, 0], bmax)
        s2_ref[0, 0] = s2_ref[0, 0] + bsum

    @pl.when(i == ng - 1)
    def _():
        t3 = _leaky(
            jnp.dot(c_ref[...], tw1_ref[...].T, preferred_element_type=jnp.float32)
            + tb1_ref[...])
        q3 = jax.nn.sigmoid(
            jnp.dot(t3, tw2_ref[...].T, preferred_element_type=jnp.float32)
            + tb2_ref[...])
        p3m = jnp.max(q3)
        p3s = jnp.sum(q3)
        out_ref[0, 0] = ((m1_ref[0, 0] / s1_ref[0, 0])
                         * (m2_ref[0, 0] / s2_ref[0, 0])
                         * (p3m / p3s))


def _heads23(sidx, h3, colsum, maxv, sumv, w1, b1, w2, b2, tw1, tb1, tw2, tb2, n):
    grid_spec = pltpu.PrefetchScalarGridSpec(
        num_scalar_prefetch=1,
        grid=(n // BN,),
        in_specs=[
            pl.BlockSpec((BN, 32), lambda i, s: (i, 0)),
            pl.BlockSpec((8, 32), lambda i, s: (s[0] // 8, 0)),
            pl.BlockSpec((1, 32), lambda i, s: (0, 0)),
            pl.BlockSpec(memory_space=pltpu.SMEM),
            pl.BlockSpec(memory_space=pltpu.SMEM),
            pl.BlockSpec((64, 64), lambda i, s: (0, 0)),
            pl.BlockSpec((1, 64), lambda i, s: (0, 0)),
            pl.BlockSpec((1, 64), lambda i, s: (0, 0)),
            pl.BlockSpec(memory_space=pltpu.SMEM),
            pl.BlockSpec((64, 32), lambda i, s: (0, 0)),
            pl.BlockSpec((1, 64), lambda i, s: (0, 0)),
            pl.BlockSpec((2, 64), lambda i, s: (0, 0)),
            pl.BlockSpec((1, 2), lambda i, s: (0, 0)),
        ],
        out_specs=[
            pl.BlockSpec(memory_space=pltpu.SMEM),
            pl.BlockSpec(memory_space=pltpu.SMEM),
            pl.BlockSpec(memory_space=pltpu.SMEM),
        ],
    )
    return pl.pallas_call(
        _k5_body,
        grid_spec=grid_spec,
        out_shape=[
            jax.ShapeDtypeStruct((1, 1), jnp.float32),
            jax.ShapeDtypeStruct((1, 1), jnp.float32),
            jax.ShapeDtypeStruct((1, 1), jnp.float32),
        ],
    )(sidx, h3, h3, colsum, maxv, sumv, w1, b1, w2, b2, tw1, tb1, tw2, tb2)


def kernel(x, edge_index, batch, Wrel0, brel0, Wroot0, Wrel1, brel1, Wroot1,
           Wrel2, brel2, Wroot2, fp_w1, fp_b1, fp_w2, fp_b2, sp_w1, sp_b1,
           sp_w2, sp_b2, tp_w1, tp_b1, tp_w2, tp_b2):
    n = x.shape[0]
    e = edge_index.shape[1]
    assert e % 128 == 0 and n % NS == 0
    rows = e // 128
    rows_pad = NW * K * ((rows + NW * K - 1) // (NW * K))
    pad = rows_pad - rows

    src2d = jnp.concatenate(
        [edge_index[0].reshape(rows, 128),
         jnp.full((pad, 128), n, jnp.int32)], axis=0)
    dst2d = jnp.concatenate(
        [edge_index[1].reshape(rows, 128),
         jnp.zeros((pad, 128), jnp.int32)], axis=0)

    sp_z = n // NS
    zch = next(k for k in range(8, 512) if sp_z % k == 0 and sp_z // k <= 128)
    zeros_in = jnp.zeros((sp_z // zch, 16), jnp.float32)

    seg1 = _make_segsum(1, n, rows_pad)
    seg4 = _make_segsum(4, n, rows_pad)

    x16 = jnp.pad(x, ((0, 1), (0, 15)))
    p0 = seg1(src2d, dst2d, x16, zeros_in)
    h1tabs = _layer0(p0, x, Wrel0.T, Wroot0.T, brel0.reshape(1, 64), n)

    p1 = seg4(src2d, dst2d, *h1tabs, zeros_in)
    h2tabs = _conv(p1, h1tabs, Wrel1, Wroot1, brel1.reshape(1, 64), n)

    p2 = seg4(src2d, dst2d, *h2tabs, zeros_in)
    h3, maxv, maxi, sumv, colsum = _conv_last(
        p2, h2tabs, Wrel2, Wroot2, brel2.reshape(1, 32),
        fp_w1, fp_b1.reshape(1, 64), fp_w2, fp_b2.reshape(1, 1), n)

    out, _, _ = _heads23(
        maxi.reshape(-1), h3, colsum, maxv, sumv,
        sp_w1, sp_b1.reshape(1, 64), sp_w2, sp_b2.reshape(1, 1),
        tp_w1, tp_b1.reshape(1, 64), tp_w2, tp_b2.reshape(1, 2), n)
    return out[0, 0]
